# baseline (speedup 1.0000x reference)
import functools

import jax
import jax.numpy as jnp
from jax import lax
from jax.experimental import pallas as pl
from jax.experimental.pallas import tpu as pltpu
from jax.experimental.pallas import tpu_sc as plsc

VOCAB = 1000000
EMBED = 32
BATCH = 4096
SEQ = 200
SH = SEQ // 2

NC = 2
NS = 16
NW = NC * NS
B_HALF = BATCH * SH
ROWS_PW = B_HALF // NW
IPS = 128
K = 10
RPB = K * IPS
NBLK = ROWS_PW // RPB
NROWCH = ROWS_PW // IPS


@functools.partial(
    pl.kernel,
    out_type=jax.ShapeDtypeStruct((B_HALF, EMBED), jnp.float32),
    mesh=plsc.VectorSubcoreMesh(core_axis_name="c", subcore_axis_name="s"),
    scratch_types=[
        pltpu.VMEM((NROWCH, IPS), jnp.int32),
        pltpu.VMEM((RPB, EMBED), jnp.float32),
        pltpu.SemaphoreType.DMA,
    ],
    compiler_params=pltpu.CompilerParams(use_tc_tiling_on_sc=False),
)
def _emb_lookup(x_hbm, table_hbm, out_hbm, idx_v, rows_v, gsem):
    wid = lax.axis_index("s") * NC + lax.axis_index("c")
    pltpu.sync_copy(x_hbm.at[wid], idx_v)
    out_base = wid * ROWS_PW

    def blk_body(blk, carry):
        descs = [
            pltpu.async_copy(
                table_hbm.at[idx_v.at[blk * K + j]],
                rows_v.at[pl.ds(j * IPS, IPS)],
                gsem,
            )
            for j in range(K)
        ]
        for d in descs:
            d.wait()
        pltpu.sync_copy(rows_v, out_hbm.at[pl.ds(out_base + blk * RPB, RPB)])
        return carry

    lax.fori_loop(0, NBLK, blk_body, 0)


def kernel(x, table):
    xi = x.astype(jnp.int32)
    halves = []
    for h in range(2):
        xh = xi[:, h * SH:(h + 1) * SH].reshape(NW, NROWCH, IPS)
        oh = _emb_lookup(xh, table)
        halves.append(oh.reshape(BATCH, SH, EMBED))
    return jnp.concatenate(halves, axis=1)


# speedup vs baseline: 1.3686x; 1.3686x over previous
import functools

import jax
import jax.numpy as jnp
from jax import lax
from jax.experimental import pallas as pl
from jax.experimental.pallas import tpu as pltpu
from jax.experimental.pallas import tpu_sc as plsc

VOCAB = 1000000
EMBED = 32

NC = 2
NS = 16
NW = NC * NS
B_TOTAL = 4096 * 200
ROWS_PW = B_TOTAL // NW
IPS = 128
K = 10
RPB = K * IPS
NBLK = ROWS_PW // RPB
NROWCH = ROWS_PW // IPS


@functools.partial(
    pl.kernel,
    out_type=jax.ShapeDtypeStruct((B_TOTAL, EMBED), jnp.float32),
    mesh=plsc.VectorSubcoreMesh(core_axis_name="c", subcore_axis_name="s"),
    scratch_types=[
        pltpu.VMEM((NROWCH, IPS), jnp.int32),
        pltpu.VMEM((RPB, EMBED), jnp.float32),
        pltpu.SemaphoreType.DMA,
    ],
    compiler_params=pltpu.CompilerParams(use_tc_tiling_on_sc=False),
)
def _emb_lookup(x_hbm, table_hbm, out_hbm, idx_v, rows_v, gsem):
    wid = lax.axis_index("s") * NC + lax.axis_index("c")
    pltpu.sync_copy(x_hbm.at[wid], idx_v)
    out_base = wid * ROWS_PW

    def blk_body(blk, carry):
        descs = [
            pltpu.async_copy(
                table_hbm.at[idx_v.at[blk * K + j]],
                rows_v.at[pl.ds(j * IPS, IPS)],
                gsem,
            )
            for j in range(K)
        ]
        for d in descs:
            d.wait()
        pltpu.sync_copy(rows_v, out_hbm.at[pl.ds(out_base + blk * RPB, RPB)])
        return carry

    lax.fori_loop(0, NBLK, blk_body, 0)


def kernel(x, table):
    x_r = x.reshape(NW, NROWCH, IPS).astype(jnp.int32)
    out = _emb_lookup(x_r, table)
    return out.reshape(x.shape[0], x.shape[1], EMBED)
